# Initial kernel scaffold; baseline (speedup 1.0000x reference)
import jax
import jax.numpy as jnp
from jax.experimental import pallas as pl


def kernel(x, edge_index, W1a, b1a, W1b, b1b, W2a, b2a, W2b, b2b, W3a, b3a, W3b, b3b, Wlin, blin):
    raise NotImplementedError("write your pallas kernel here")


# speedup vs baseline: 11.3032x; 11.3032x over previous
import functools

import jax
import jax.numpy as jnp
from jax import lax
from jax.experimental import pallas as pl
from jax.experimental.pallas import tpu as pltpu
from jax.experimental.pallas import tpu_sc as plsc

N = 10000
E = 320000
D = 128
H = 20
HP = 32
C = 10

NC = 2
NS = 16
NW = NC * NS
EPW = E // NW
CH = 128
NFULL = EPW // CH
TAIL = EPW - NFULL * CH
RPS = 624
RTL = N - NS * RPS


def _sc_agg_body(src_hbm, dst_hbm, y_hbm, zeros_hbm, out_hbm,
                 src_a, dst_a, rows_a, src_b, dst_b, rows_b,
                 tsrc, tdst, trows, agg_s, sem_a, sem_b):
    c = lax.axis_index("c")
    s = lax.axis_index("s")
    wid = c * NS + s
    base = wid * EPW

    pltpu.sync_copy(zeros_hbm.at[pl.ds(s * RPS, RPS)],
                    agg_s.at[pl.ds(s * RPS, RPS)])

    @pl.when(s == 0)
    def _():
        pltpu.sync_copy(zeros_hbm.at[pl.ds(NS * RPS, RTL)],
                        agg_s.at[pl.ds(NS * RPS, RTL)])
    plsc.subcore_barrier()

    pltpu.sync_copy(src_hbm.at[pl.ds(base, CH)], src_a)
    pltpu.sync_copy(dst_hbm.at[pl.ds(base, CH)], dst_a)
    gat_a = pltpu.async_copy(y_hbm.at[src_a], rows_a, sem_a)

    def chunk(j, carry):
        even = j % 2 == 0
        off = base + (j + 1) * CH

        def do(cur_src, cur_dst, cur_rows, cur_gat_sem,
               nxt_src, nxt_dst, nxt_rows, nxt_sem):
            @pl.when(j + 1 < NFULL)
            def _():
                pltpu.sync_copy(src_hbm.at[pl.ds(off, CH)], nxt_src)
                pltpu.sync_copy(dst_hbm.at[pl.ds(off, CH)], nxt_dst)
                pltpu.async_copy(y_hbm.at[nxt_src], nxt_rows, nxt_sem)
            pltpu.make_async_copy(y_hbm.at[cur_src], cur_rows, cur_gat_sem).wait()
            pltpu.sync_copy(cur_rows, agg_s.at[cur_dst], add=True)

        @pl.when(even)
        def _():
            do(src_a, dst_a, rows_a, sem_a, src_b, dst_b, rows_b, sem_b)

        @pl.when(jnp.logical_not(even))
        def _():
            do(src_b, dst_b, rows_b, sem_b, src_a, dst_a, rows_a, sem_a)
        return carry

    lax.fori_loop(0, NFULL, chunk, 0)

    toff = base + NFULL * CH
    pltpu.sync_copy(src_hbm.at[pl.ds(toff, TAIL)], tsrc)
    pltpu.sync_copy(dst_hbm.at[pl.ds(toff, TAIL)], tdst)
    pltpu.async_copy(y_hbm.at[tsrc], trows, sem_a).wait()
    pltpu.sync_copy(trows, agg_s.at[tdst], add=True)

    plsc.subcore_barrier()
    pltpu.sync_copy(agg_s.at[pl.ds(s * RPS, RPS)],
                    out_hbm.at[c, pl.ds(s * RPS, RPS)])

    @pl.when(s == 0)
    def _():
        pltpu.sync_copy(agg_s.at[pl.ds(NS * RPS, RTL)],
                        out_hbm.at[c, pl.ds(NS * RPS, RTL)])


_sc_agg = functools.partial(
    pl.kernel,
    out_type=jax.ShapeDtypeStruct((NC, N, HP), jnp.float32),
    mesh=plsc.VectorSubcoreMesh(core_axis_name="c", subcore_axis_name="s"),
    scratch_types=[
        pltpu.VMEM((CH,), jnp.int32),
        pltpu.VMEM((CH,), jnp.int32),
        pltpu.VMEM((CH, HP), jnp.float32),
        pltpu.VMEM((CH,), jnp.int32),
        pltpu.VMEM((CH,), jnp.int32),
        pltpu.VMEM((CH, HP), jnp.float32),
        pltpu.VMEM((TAIL,), jnp.int32),
        pltpu.VMEM((TAIL,), jnp.int32),
        pltpu.VMEM((TAIL, HP), jnp.float32),
        pltpu.VMEM_SHARED((N, HP), jnp.float32),
        pltpu.SemaphoreType.DMA,
        pltpu.SemaphoreType.DMA,
    ],
    compiler_params=pltpu.CompilerParams(use_tc_tiling_on_sc=False),
)(_sc_agg_body)


ROWB = 1000
GRID = N // ROWB


def _proj_body(x_ref, w_ref, o_ref):
    o_ref[...] = jnp.dot(x_ref[...], w_ref[...],
                         preferred_element_type=jnp.float32)


_proj = pl.pallas_call(
    _proj_body,
    grid=(GRID,),
    in_specs=[pl.BlockSpec((ROWB, D), lambda i: (i, 0)),
              pl.BlockSpec((D, HP), lambda i: (0, 0))],
    out_specs=pl.BlockSpec((ROWB, HP), lambda i: (i, 0)),
    out_shape=jax.ShapeDtypeStruct((N, HP), jnp.float32),
)


def _mlp_rows(p_ref, ba_ref, wb_ref, bb_ref):
    a = p_ref[0] + p_ref[1]
    t = jnp.maximum(a + ba_ref[...], 0.0)
    u = jnp.maximum(
        jnp.dot(t, wb_ref[...], preferred_element_type=jnp.float32)
        + bb_ref[...], 0.0)
    nrm = jnp.maximum(jnp.sqrt(jnp.sum(u * u, axis=1, keepdims=True)), 1e-12)
    return jnp.maximum(u / nrm, 0.0)


def _mid_body(p_ref, ba_ref, wb_ref, bb_ref, wn_ref, o_ref):
    h = _mlp_rows(p_ref, ba_ref, wb_ref, bb_ref)
    o_ref[...] = jnp.dot(h, wn_ref[...], preferred_element_type=jnp.float32)


_mid = pl.pallas_call(
    _mid_body,
    grid=(GRID,),
    in_specs=[pl.BlockSpec((NC, ROWB, HP), lambda i: (0, i, 0)),
              pl.BlockSpec((1, HP), lambda i: (0, 0)),
              pl.BlockSpec((HP, HP), lambda i: (0, 0)),
              pl.BlockSpec((1, HP), lambda i: (0, 0)),
              pl.BlockSpec((HP, HP), lambda i: (0, 0))],
    out_specs=pl.BlockSpec((ROWB, HP), lambda i: (i, 0)),
    out_shape=jax.ShapeDtypeStruct((N, HP), jnp.float32),
)


def _fin_body(p_ref, ba_ref, wb_ref, bb_ref, wl1_ref, wl2_ref, bl_ref, o_ref):
    h = _mlp_rows(p_ref, ba_ref, wb_ref, bb_ref)
    hmax = jnp.max(h, axis=0, keepdims=True)
    hmean = jnp.sum(h, axis=0, keepdims=True) * (1.0 / N)
    o_ref[...] = (
        jnp.dot(hmax, wl1_ref[...], preferred_element_type=jnp.float32)
        + jnp.dot(hmean, wl2_ref[...], preferred_element_type=jnp.float32)
        + bl_ref[...])


_fin = pl.pallas_call(
    _fin_body,
    out_shape=jax.ShapeDtypeStruct((1, C), jnp.float32),
)


def _pad_w(w):
    fi, fo = w.shape
    return jnp.pad(w, ((0, HP - fi if fi == H else 0), (0, HP - fo)))


def _pad_b(b):
    return jnp.pad(b, (0, HP - H)).reshape(1, HP)


def kernel(x, edge_index, W1a, b1a, W1b, b1b, W2a, b2a, W2b, b2b,
           W3a, b3a, W3b, b3b, Wlin, blin):
    src = edge_index[0]
    dst = edge_index[1]
    zeros = jnp.zeros((N, HP), jnp.float32)

    W1a_p = jnp.pad(W1a, ((0, 0), (0, HP - H)))
    W1b_p, W2a_p, W2b_p, W3a_p, W3b_p = map(_pad_w, (W1b, W2a, W2b, W3a, W3b))
    b1a_p, b1b_p, b2a_p, b2b_p, b3a_p, b3b_p = map(
        _pad_b, (b1a, b1b, b2a, b2b, b3a, b3b))
    wl1 = jnp.pad(Wlin[:H], ((0, HP - H), (0, 0)))
    wl2 = jnp.pad(Wlin[H:], ((0, HP - H), (0, 0)))
    bl = blin.reshape(1, C)

    y1 = _proj(x, W1a_p)
    p1 = _sc_agg(src, dst, y1, zeros)
    y2 = _mid(p1, b1a_p, W1b_p, b1b_p, W2a_p)
    p2 = _sc_agg(src, dst, y2, zeros)
    y3 = _mid(p2, b2a_p, W2b_p, b2b_p, W3a_p)
    p3 = _sc_agg(src, dst, y3, zeros)
    return _fin(p3, b3a_p, W3b_p, b3b_p, wl1, wl2, bl)
